# baseline (speedup 1.0000x reference)
import jax
import jax.numpy as jnp
from jax.experimental import pallas as pl
from jax.experimental.pallas import tpu as pltpu

_B, _N = 16, 8192
_IN, _H, _EMB, _HA, _OUT = 64, 128, 64, 128, 64
_NUM_AISLES = 32
_G = 4


def _lrelu(v):
    return jnp.maximum(v, v * jnp.asarray(0.01, v.dtype))


def _fused_kernel(x_ref, ids_ref, mask_ref,
                  w1_ref, b1_ref, w2_ref, b2_ref, w3_ref, b3_ref,
                  w4_ref, b4_ref, w5_ref, b5_ref, w6_ref, b6_ref,
                  out_ref):
    f32, bf = jnp.float32, jnp.bfloat16
    b = pl.program_id(0)
    w1 = w1_ref[...].astype(bf)
    w2 = w2_ref[...].astype(bf)
    w3 = w3_ref[...].astype(bf)
    w4 = w4_ref[...].astype(bf)
    w5 = w5_ref[...].astype(bf)
    w6 = w6_ref[...].astype(bf)
    xb = x_ref[...].astype(bf)
    h = _lrelu(jnp.dot(xb, w1, preferred_element_type=f32).astype(bf)
               + b1_ref[...].astype(bf)[None, :])
    h = _lrelu(jnp.dot(h, w2, preferred_element_type=f32).astype(bf)
               + b2_ref[...].astype(bf)[None, :])
    zb = (jnp.dot(h, w3, preferred_element_type=f32).astype(bf)
          + b3_ref[...].astype(bf)[None, :])

    nrow = _G * _N
    ncls = _G * _NUM_AISLES
    ids = ids_ref[...][None, :]
    ids = ids + _NUM_AISLES * (
        jax.lax.broadcasted_iota(jnp.int32, (1, nrow), 1) // _N)
    ids16 = ids.astype(jnp.int16)
    oh = (jnp.broadcast_to(ids16, (ncls, nrow)) ==
          jax.lax.broadcasted_iota(jnp.int16, (ncls, nrow), 0)).astype(bf)
    z1 = jnp.concatenate([zb, jnp.ones((nrow, 1), bf)], axis=1)
    sc = jax.lax.dot_general(oh, z1, (((1,), (0,)), ((), ())),
                             preferred_element_type=f32)
    sums, counts = sc[:, :_EMB], sc[:, _EMB:]
    means = (sums / jnp.maximum(counts, 1.0)).astype(bf)
    g = jax.lax.dot_general(oh, means, (((0,), (0,)), ((), ())),
                            preferred_element_type=f32)

    cat = jnp.concatenate([zb, g.astype(bf)], axis=1)
    h2 = _lrelu(jnp.dot(cat, w4, preferred_element_type=f32).astype(bf)
                + b4_ref[...].astype(bf)[None, :])
    h2 = _lrelu(jnp.dot(h2, w5, preferred_element_type=f32).astype(bf)
                + b5_ref[...].astype(bf)[None, :])
    scores = jax.lax.dot_general(w6, h2, (((0,), (1,)), ((), ())),
                                 preferred_element_type=f32) + b6_ref[0]

    for gi in range(_G):
        row = _G * b + gi
        s = scores[:, gi * _N:(gi + 1) * _N]
        mk = mask_ref[pl.ds(row, 1), :]
        logits = jnp.where(mk != 0, s, -jnp.inf)
        mx = jnp.max(logits, axis=1, keepdims=True)
        e = jnp.exp(logits - mx)
        out_ref[pl.ds(row, 1), :] = e / jnp.sum(e, axis=1, keepdims=True)


def kernel(x, aisle_nrs, mask, W1, b1, W2, b2, W3, b3, W4, b4, W5, b5, W6, b6):
    ids = aisle_nrs.astype(jnp.int32)

    full = lambda arr: pl.BlockSpec(arr.shape, lambda b: (0,) * arr.ndim)
    weights = [W1, b1, W2, b2, W3, b3, W4, b4, W5, b5, W6, b6]

    probs = pl.pallas_call(
        _fused_kernel,
        grid=(_B // _G,),
        in_specs=[pl.BlockSpec((_G * _N, _IN), lambda b: (b, 0)),
                  pl.BlockSpec((_G * _N,), lambda b: (b,)),
                  full(mask)] + [full(w) for w in weights],
        out_specs=pl.BlockSpec((_B, _N), lambda b: (0, 0)),
        out_shape=jax.ShapeDtypeStruct((_B, _N), jnp.float32),
        compiler_params=pltpu.CompilerParams(
            dimension_semantics=("arbitrary",)),
    )(x, ids, mask, *weights)

    return probs


# speedup vs baseline: 1.0428x; 1.0428x over previous
import jax
import jax.numpy as jnp
from jax.experimental import pallas as pl
from jax.experimental.pallas import tpu as pltpu

_B, _N = 16, 8192
_IN, _H, _EMB, _HA, _OUT = 64, 128, 64, 128, 64
_NUM_AISLES = 32
_G = 4


def _lrelu(v):
    return jnp.maximum(v, v * jnp.asarray(0.01, v.dtype))


def _fused_kernel(x_ref, ids_ref, mask_ref,
                  w1_ref, b1_ref, w2_ref, b2_ref, w3_ref, b3_ref,
                  w4_ref, b4_ref, w5_ref, b5_ref, w6_ref, b6_ref,
                  out_ref):
    f32, bf = jnp.float32, jnp.bfloat16
    b = pl.program_id(0)
    w1 = w1_ref[...].astype(bf)
    w2 = w2_ref[...].astype(bf)
    w3 = w3_ref[...].astype(bf)
    w4 = w4_ref[...].astype(bf)
    w5 = w5_ref[...].astype(bf)
    w6 = w6_ref[...].astype(bf)
    xb = x_ref[...].astype(bf)
    h = _lrelu(jnp.dot(xb, w1, preferred_element_type=f32).astype(bf)
               + b1_ref[...].astype(bf)[None, :])
    h = _lrelu(jnp.dot(h, w2, preferred_element_type=f32).astype(bf)
               + b2_ref[...].astype(bf)[None, :])
    zb = (jnp.dot(h, w3, preferred_element_type=f32).astype(bf)
          + b3_ref[...].astype(bf)[None, :])

    nrow = _G * _N
    ncls = _G * _NUM_AISLES
    ids = ids_ref[...][None, :]
    ids = ids + _NUM_AISLES * (
        jax.lax.broadcasted_iota(jnp.int32, (1, nrow), 1) // _N)
    oh = (jnp.broadcast_to(ids, (ncls, nrow)) ==
          jax.lax.broadcasted_iota(jnp.int32, (ncls, nrow), 0)).astype(bf)
    z1 = jnp.concatenate([zb, jnp.ones((nrow, 1), bf)], axis=1)
    sc = jax.lax.dot_general(oh, z1, (((1,), (0,)), ((), ())),
                             preferred_element_type=f32)
    sums, counts = sc[:, :_EMB], sc[:, _EMB:]
    means = (sums / jnp.maximum(counts, 1.0)).astype(bf)
    g = jax.lax.dot_general(oh, means, (((0,), (0,)), ((), ())),
                            preferred_element_type=f32)

    cat = jnp.concatenate([zb, g.astype(bf)], axis=1)
    h2 = _lrelu(jnp.dot(cat, w4, preferred_element_type=f32).astype(bf)
                + b4_ref[...].astype(bf)[None, :])
    h2 = _lrelu(jnp.dot(h2, w5, preferred_element_type=f32).astype(bf)
                + b5_ref[...].astype(bf)[None, :])
    scores = jax.lax.dot_general(w6, h2, (((0,), (1,)), ((), ())),
                                 preferred_element_type=f32) + b6_ref[0]

    for gi in range(_G):
        row = _G * b + gi
        s = scores[:, gi * _N:(gi + 1) * _N]
        mk = mask_ref[pl.ds(row, 1), :]
        logits = jnp.where(mk != 0, s, -jnp.inf)
        mx = jnp.max(logits, axis=1, keepdims=True)
        e = jnp.exp(logits - mx)
        out_ref[pl.ds(row, 1), :] = e / jnp.sum(e, axis=1, keepdims=True)


def kernel(x, aisle_nrs, mask, W1, b1, W2, b2, W3, b3, W4, b4, W5, b5, W6, b6):
    ids = aisle_nrs.astype(jnp.int32)

    full = lambda arr: pl.BlockSpec(arr.shape, lambda b: (0,) * arr.ndim)
    weights = [W1, b1, W2, b2, W3, b3, W4, b4, W5, b5, W6, b6]

    probs = pl.pallas_call(
        _fused_kernel,
        grid=(_B // _G,),
        in_specs=[pl.BlockSpec((_G * _N, _IN), lambda b: (b, 0)),
                  pl.BlockSpec((_G * _N,), lambda b: (b,)),
                  full(mask)] + [full(w) for w in weights],
        out_specs=pl.BlockSpec((_B, _N), lambda b: (0, 0)),
        out_shape=jax.ShapeDtypeStruct((_B, _N), jnp.float32),
        compiler_params=pltpu.CompilerParams(
            dimension_semantics=("arbitrary",)),
    )(x, ids, mask, *weights)

    return probs
